# baseline (speedup 1.0000x reference)
import functools

import jax
import jax.numpy as jnp
from jax import lax
from jax.experimental import pallas as pl
from jax.experimental.pallas import tpu as pltpu
from jax.experimental.pallas import tpu_sc as plsc

_NC = 2
_NS = 16
_NW = _NC * _NS

_T1 = 1024
_C2 = 1024


def _make_transpose(V, D):
    n_full = V // _T1
    tail = V - n_full * _T1
    mesh = plsc.VectorSubcoreMesh(core_axis_name="c", subcore_axis_name="s")

    @functools.partial(
        pl.kernel,
        mesh=mesh,
        out_type=jax.ShapeDtypeStruct((V, D), jnp.float32),
        scratch_types=[
            pltpu.VMEM((D, _T1), jnp.float32),
            pltpu.VMEM((_T1, D), jnp.float32),
        ],
        compiler_params=pltpu.CompilerParams(use_tc_tiling_on_sc=False, needs_layout_passes=False),
    )
    def transpose(wt_hbm, wr_hbm, inb, outb):
        wid = lax.axis_index("s") * _NC + lax.axis_index("c")
        iota = lax.iota(jnp.int32, 16)
        f_lo = iota
        f_hi = iota + 16

        def transpose_rows(n_rows):
            def body(u, carry):
                u_vec = jnp.full((16,), u, jnp.int32)
                lo = plsc.load_gather(inb, [f_lo, u_vec])
                hi = plsc.load_gather(inb, [f_hi, u_vec])
                outb[u, pl.ds(0, 16)] = lo
                outb[u, pl.ds(16, 16)] = hi
                return carry

            lax.fori_loop(0, n_rows, body, 0, unroll=8)

        def slab_loop(s, carry):
            g = wid + s * _NW
            v0 = g * _T1
            pltpu.sync_copy(wt_hbm.at[:, pl.ds(v0, _T1)], inb)
            transpose_rows(_T1)
            pltpu.sync_copy(outb, wr_hbm.at[pl.ds(v0, _T1), :])
            return carry

        n_my = jnp.where(wid < (n_full % _NW), n_full // _NW + 1, n_full // _NW)
        lax.fori_loop(0, n_my, slab_loop, 0)

        if tail:
            @pl.when(wid == _NW - 1)
            def _():
                pltpu.sync_copy(
                    wt_hbm.at[:, pl.ds(n_full * _T1, tail)],
                    inb.at[:, pl.ds(0, tail)],
                )
                transpose_rows(tail)
                pltpu.sync_copy(
                    outb.at[pl.ds(0, tail), :],
                    wr_hbm.at[pl.ds(n_full * _T1, tail), :],
                )

    return transpose


def _make_gather(V, D, L, B):
    n_win = B // _C2
    n_chunks = L * n_win // _NW
    mesh = plsc.VectorSubcoreMesh(core_axis_name="c", subcore_axis_name="s")

    @functools.partial(
        pl.kernel,
        mesh=mesh,
        out_type=jax.ShapeDtypeStruct(
            (L, D // 8, B // 128, 8, 128), jnp.float32
        ),
        scratch_types=[
            pltpu.VMEM((_C2,), jnp.int32),
            pltpu.VMEM((_C2, D), jnp.float32),
            pltpu.VMEM((D // 8, _C2 // 128, 8, 128), jnp.float32),
            pltpu.SemaphoreType.DMA,
        ],
        compiler_params=pltpu.CompilerParams(use_tc_tiling_on_sc=False, needs_layout_passes=False),
    )
    def gather(wr_hbm, idx_hbm, out_hbm, idxb, rows, outt, sem):
        wid = lax.axis_index("s") * _NC + lax.axis_index("c")
        bw = (wid % n_win) * _C2
        l0 = wid // n_win
        iota = lax.iota(jnp.int32, 16)
        f_vecs = [jnp.full((16,), f, jnp.int32) for f in range(D)]

        def chunk(k, carry):
            l = l0 + 2 * k
            pltpu.sync_copy(idx_hbm.at[pl.ds(l * B + bw, _C2)], idxb)
            pltpu.async_copy(wr_hbm.at[idxb], rows, sem).wait()

            def tpose(j, carry):
                j0 = j * 16
                ridx = iota + j0
                bt = j0 // 128
                bi = j0 % 128
                for f in range(D):
                    vals = plsc.load_gather(rows, [ridx, f_vecs[f]])
                    outt[f // 8, bt, f % 8, pl.ds(bi, 16)] = vals
                return carry

            lax.fori_loop(0, _C2 // 16, tpose, 0)
            pltpu.sync_copy(
                outt, out_hbm.at[l, :, pl.ds(bw // 128, _C2 // 128), :, :]
            )
            return carry

        lax.fori_loop(0, n_chunks, chunk, 0)

    return gather


def kernel(word_indexes, W):
    B, L = word_indexes.shape
    V, D = W.shape
    wt = W.T
    idx_flat = word_indexes.T.astype(jnp.int32).reshape(B * L)
    wr = _make_transpose(V, D)(wt)
    out5 = _make_gather(V, D, L, B)(wr, idx_flat)
    return out5.transpose(2, 4, 0, 1, 3).reshape(B, L, D)


# speedup vs baseline: 3.3048x; 3.3048x over previous
import functools

import jax
import jax.numpy as jnp
from jax import lax
from jax.experimental import pallas as pl
from jax.experimental.pallas import tpu as pltpu
from jax.experimental.pallas import tpu_sc as plsc

_NC = 2
_NS = 16
_NW = _NC * _NS

_T1 = 1024
_C2 = 1024


def _make_detile(V, D):
    n_full = V // _T1
    tail = V - n_full * _T1
    slab = _T1 * D
    mesh = plsc.VectorSubcoreMesh(core_axis_name="c", subcore_axis_name="s")

    @functools.partial(
        pl.kernel,
        mesh=mesh,
        out_type=jax.ShapeDtypeStruct((V * D,), jnp.float32),
        scratch_types=[
            pltpu.VMEM((D, _T1), jnp.float32),
            pltpu.VMEM((tail * D,), jnp.float32),
            pltpu.SemaphoreType.DMA,
        ],
    )
    def detile(wt_hbm, tail_hbm, blk_hbm, inb, tailb, sem):
        wid = lax.axis_index("s") * _NC + lax.axis_index("c")

        def slab_loop(s, carry):
            g = wid + s * _NW
            pltpu.sync_copy(wt_hbm.at[:, pl.ds(g * _T1, _T1)], inb)
            copies = [
                pltpu.async_copy(
                    inb.at[f, :],
                    blk_hbm.at[pl.ds(g * slab + f * _T1, _T1)],
                    sem,
                )
                for f in range(D)
            ]
            for c in copies:
                c.wait()
            return carry

        n_my = jnp.where(wid < (n_full % _NW), n_full // _NW + 1, n_full // _NW)
        lax.fori_loop(0, n_my, slab_loop, 0)

        if tail:
            @pl.when(wid == _NW - 1)
            def _():
                pltpu.sync_copy(tail_hbm, tailb)
                pltpu.sync_copy(tailb, blk_hbm.at[pl.ds(n_full * slab, tail * D)])

    return detile


def _make_transpose(V, D):
    n_full = V // _T1
    tail = V - n_full * _T1
    slab = _T1 * D
    mesh = plsc.VectorSubcoreMesh(core_axis_name="c", subcore_axis_name="s")

    @functools.partial(
        pl.kernel,
        mesh=mesh,
        out_type=jax.ShapeDtypeStruct((V * D,), jnp.float32),
        scratch_types=[
            pltpu.VMEM((slab,), jnp.float32),
            pltpu.VMEM((slab,), jnp.float32),
        ],
        compiler_params=pltpu.CompilerParams(use_tc_tiling_on_sc=False, needs_layout_passes=False),
    )
    def transpose(blk_hbm, wr_hbm, inb, outb):
        wid = lax.axis_index("s") * _NC + lax.axis_index("c")
        iota = lax.iota(jnp.int32, 16)
        fvecs = [iota * D + f for f in range(D)]

        def slab_loop(s, carry):
            g = wid + s * _NW
            pltpu.sync_copy(blk_hbm.at[pl.ds(g * slab, slab)], inb)

            def block(b, carry):
                u0 = b * 16
                base = u0 * D
                for f in range(D):
                    vals = inb[pl.ds(f * _T1 + u0, 16)]
                    plsc.store_scatter(outb, [fvecs[f] + base], vals)
                return carry

            lax.fori_loop(0, _T1 // 16, block, 0)
            pltpu.sync_copy(outb, wr_hbm.at[pl.ds(g * slab, slab)])
            return carry

        n_my = jnp.where(wid < (n_full % _NW), n_full // _NW + 1, n_full // _NW)
        lax.fori_loop(0, n_my, slab_loop, 0)

        if tail:
            @pl.when(wid == _NW - 1)
            def _():
                pltpu.sync_copy(
                    blk_hbm.at[pl.ds(n_full * slab, tail * D)],
                    inb.at[pl.ds(0, tail * D)],
                )
                pltpu.sync_copy(
                    inb.at[pl.ds(0, tail * D)],
                    wr_hbm.at[pl.ds(n_full * slab, tail * D)],
                )

    return transpose


def _make_gather(V, D, L, B):
    n_win = B // _C2
    n_chunks = L * n_win // _NW
    seg = 8 * _C2
    mesh = plsc.VectorSubcoreMesh(core_axis_name="c", subcore_axis_name="s")

    @functools.partial(
        pl.kernel,
        mesh=mesh,
        out_type=jax.ShapeDtypeStruct((L, B * D), jnp.float32),
        scratch_types=[
            pltpu.VMEM((_C2,), jnp.int32),
            pltpu.VMEM((_C2, D), jnp.float32),
            pltpu.VMEM((D // 8 * seg,), jnp.float32),
            pltpu.SemaphoreType.DMA,
        ],
        compiler_params=pltpu.CompilerParams(use_tc_tiling_on_sc=False, needs_layout_passes=False),
    )
    def gather(wr_hbm, idx_hbm, out_hbm, idxb, rows, outt, sem):
        wid = lax.axis_index("s") * _NC + lax.axis_index("c")
        bw = (wid % n_win) * _C2
        l0 = wid // n_win
        iota = lax.iota(jnp.int32, 16)
        svec_lo = ((iota // 8) * seg) + ((iota % 8) * 128)
        svec_hi = svec_lo + 2 * seg

        def chunk(k, carry):
            l = l0 + 2 * k
            pltpu.sync_copy(idx_hbm.at[pl.ds(l * B + bw, _C2)], idxb)
            pltpu.async_copy(wr_hbm.at[idxb], rows, sem).wait()

            def tpose(j, carry):
                base = (j // 128) * 1024 + (j % 128)
                lo = rows[j, pl.ds(0, 16)]
                hi = rows[j, pl.ds(16, 16)]
                plsc.store_scatter(outt, [svec_lo + base], lo)
                plsc.store_scatter(outt, [svec_hi + base], hi)
                return carry

            lax.fori_loop(0, _C2, tpose, 0, unroll=8)
            for ft in range(D // 8):
                pltpu.sync_copy(
                    outt.at[pl.ds(ft * seg, seg)],
                    out_hbm.at[l, pl.ds(ft * (B * 8) + bw * 8, seg)],
                )
            return carry

        lax.fori_loop(0, n_chunks, chunk, 0)

    return gather


def kernel(word_indexes, W):
    B, L = word_indexes.shape
    V, D = W.shape
    wt = W.T
    n_full = V // _T1
    tail = V - n_full * _T1
    tail_lin = W[n_full * _T1:, :].reshape(tail * D)
    idx_flat = word_indexes.T.astype(jnp.int32).reshape(B * L)
    blk = _make_detile(V, D)(wt, tail_lin)
    wr = _make_transpose(V, D)(blk)
    out2 = _make_gather(V, D, L, B)(wr.reshape(V, D), idx_flat)
    out5 = out2.reshape(L, D // 8, B // 128, 8, 128)
    return out5.transpose(2, 4, 0, 1, 3).reshape(B, L, D)


# speedup vs baseline: 6.9807x; 2.1123x over previous
import functools

import jax
import jax.numpy as jnp
from jax import lax
from jax.experimental import pallas as pl
from jax.experimental.pallas import tpu as pltpu
from jax.experimental.pallas import tpu_sc as plsc

_NC = 2
_NS = 16
_NW = _NC * _NS

_T1 = 1024
_C2 = 1024

_SC_PARAMS = pltpu.CompilerParams(
    use_tc_tiling_on_sc=False, needs_layout_passes=False
)


def _make_detile(V, D):
    n_full = V // _T1
    tail = V - n_full * _T1
    slab = _T1 * D
    mesh = plsc.VectorSubcoreMesh(core_axis_name="c", subcore_axis_name="s")

    @functools.partial(
        pl.kernel,
        mesh=mesh,
        out_type=jax.ShapeDtypeStruct((n_full * slab,), jnp.float32),
        scratch_types=[
            pltpu.VMEM((D, _T1), jnp.float32),
            pltpu.SemaphoreType.DMA,
        ],
    )
    def detile(wt_hbm, blk_hbm, inb, sem):
        wid = lax.axis_index("s") * _NC + lax.axis_index("c")

        def slab_loop(s, carry):
            g = wid + s * _NW
            pltpu.sync_copy(wt_hbm.at[:, pl.ds(g * _T1, _T1)], inb)
            copies = [
                pltpu.async_copy(
                    inb.at[f, :],
                    blk_hbm.at[pl.ds(g * slab + f * _T1, _T1)],
                    sem,
                )
                for f in range(D)
            ]
            for c in copies:
                c.wait()
            return carry

        n_my = jnp.where(wid < (n_full % _NW), n_full // _NW + 1, n_full // _NW)
        lax.fori_loop(0, n_my, slab_loop, 0)

    return detile


def _make_transpose(V, D):
    n_full = V // _T1
    tail = V - n_full * _T1
    slab = _T1 * D
    mesh = plsc.VectorSubcoreMesh(core_axis_name="c", subcore_axis_name="s")

    @functools.partial(
        pl.kernel,
        mesh=mesh,
        out_type=jax.ShapeDtypeStruct((V, D), jnp.float32),
        scratch_types=[
            pltpu.VMEM((D, _T1), jnp.float32),
            pltpu.VMEM((_T1, D), jnp.float32),
            pltpu.VMEM((tail, D), jnp.float32),
            pltpu.SemaphoreType.DMA,
        ],
        compiler_params=_SC_PARAMS,
    )
    def transpose(blk_hbm, tail_hbm, wr_hbm, inb, outb, tailb, sem):
        wid = lax.axis_index("s") * _NC + lax.axis_index("c")
        iota = lax.iota(jnp.int32, 16)
        iota16 = iota + 16

        def slab_loop(s, carry):
            g = wid + s * _NW
            copies = [
                pltpu.async_copy(
                    blk_hbm.at[pl.ds(g * slab + f * _T1, _T1)],
                    inb.at[f, :],
                    sem,
                )
                for f in range(D)
            ]
            for c in copies:
                c.wait()

            def body(u, carry):
                t = (u + iota) & (_T1 - 1)
                lo = plsc.load_gather(inb, [iota, t])
                hi = plsc.load_gather(inb, [iota16, t])
                plsc.store_scatter(outb, [t, iota], lo)
                plsc.store_scatter(outb, [t, iota16], hi)
                return carry

            lax.fori_loop(0, _T1, body, 0, unroll=8)
            pltpu.sync_copy(outb, wr_hbm.at[pl.ds(g * _T1, _T1), :])
            return carry

        n_my = jnp.where(wid < (n_full % _NW), n_full // _NW + 1, n_full // _NW)
        lax.fori_loop(0, n_my, slab_loop, 0)

        if tail:
            @pl.when(wid == _NW - 1)
            def _():
                pltpu.sync_copy(tail_hbm, tailb)
                pltpu.sync_copy(tailb, wr_hbm.at[pl.ds(n_full * _T1, tail), :])

    return transpose


def _make_gather(V, D, L, B):
    n_win = B // _C2
    n_chunks = L * n_win // _NW
    seg = (B // 128) * 8 * 128 // n_win
    mesh = plsc.VectorSubcoreMesh(core_axis_name="c", subcore_axis_name="s")

    @functools.partial(
        pl.kernel,
        mesh=mesh,
        out_type=jax.ShapeDtypeStruct(
            (L, D // 8, (B // 128) * 8 * 128), jnp.float32
        ),
        scratch_types=[
            pltpu.VMEM((_C2,), jnp.int32),
            pltpu.VMEM((_C2, D), jnp.float32),
            pltpu.VMEM((D // 8, seg), jnp.float32),
            pltpu.SemaphoreType.DMA,
        ],
        compiler_params=_SC_PARAMS,
    )
    def gather(wr_hbm, idx_hbm, out_hbm, idxb, rows, outt, sem):
        wid = lax.axis_index("s") * _NC + lax.axis_index("c")
        bw = (wid % n_win) * _C2
        l0 = wid // n_win
        iota = lax.iota(jnp.int32, 16)
        fvecs = [(c + iota) % D for c in range(D)]
        ftvecs = [fvecs[c] // 8 for c in range(D)]
        posvecs = [(fvecs[c] % 8) * 128 for c in range(D)]

        def chunk(k, carry):
            l = l0 + 2 * k
            pltpu.sync_copy(idx_hbm.at[pl.ds(l * B + bw, _C2)], idxb)
            pltpu.async_copy(wr_hbm.at[idxb], rows, sem).wait()

            def tpose(b, carry):
                j0 = b * 16
                jv = iota + j0
                jpart = ((jv >> 7) << 10) + (jv & 127)
                for c in range(D):
                    vals = plsc.load_gather(rows, [jv, fvecs[c]])
                    plsc.store_scatter(outt, [ftvecs[c], posvecs[c] + jpart], vals)
                return carry

            lax.fori_loop(0, _C2 // 16, tpose, 0, unroll=2)
            for ft in range(D // 8):
                pltpu.sync_copy(
                    outt.at[ft], out_hbm.at[l, ft, pl.ds(bw * 8, seg)]
                )
            return carry

        lax.fori_loop(0, n_chunks, chunk, 0)

    return gather


def kernel(word_indexes, W):
    B, L = word_indexes.shape
    V, D = W.shape
    wt = W.T
    n_full = V // _T1
    tail = V - n_full * _T1
    tail_rows = W[n_full * _T1:, :]
    idx_flat = word_indexes.T.astype(jnp.int32).reshape(B * L)
    blk = _make_detile(V, D)(wt)
    wr = _make_transpose(V, D)(blk, tail_rows)
    out3 = _make_gather(V, D, L, B)(wr, idx_flat)
    out5 = out3.reshape(L, D // 8, B // 128, 8, 128)
    return out5.transpose(2, 4, 0, 1, 3).reshape(B, L, D)


# speedup vs baseline: 7.1648x; 1.0264x over previous
import functools

import jax
import jax.numpy as jnp
from jax import lax
from jax.experimental import pallas as pl
from jax.experimental.pallas import tpu as pltpu
from jax.experimental.pallas import tpu_sc as plsc

_NC = 2
_NS = 16
_NW = _NC * _NS

_T1 = 896
_C2 = 1024

_SC_PARAMS = pltpu.CompilerParams(
    use_tc_tiling_on_sc=False, needs_layout_passes=False
)


def _n_my(wid, n_full):
    return jnp.where(
        wid < (n_full % _NW), n_full // _NW + 1, n_full // _NW
    )


def _make_detile(V, D):
    n_full = V // _T1
    mesh = plsc.VectorSubcoreMesh(core_axis_name="c", subcore_axis_name="s")

    @functools.partial(
        pl.kernel,
        mesh=mesh,
        out_type=jax.ShapeDtypeStruct((n_full * D, _T1), jnp.float32),
        scratch_types=[
            pltpu.VMEM((D, _T1), jnp.float32),
            pltpu.VMEM((D, _T1), jnp.float32),
            pltpu.SemaphoreType.DMA,
            pltpu.SemaphoreType.DMA,
        ],
    )
    def detile(wt_hbm, blk_hbm, inb0, inb1, sem_i, sem_o):
        wid = lax.axis_index("s") * _NC + lax.axis_index("c")
        n_my = _n_my(wid, n_full)
        bufs = (inb0, inb1)

        def src(s):
            return wt_hbm.at[:, pl.ds((wid + s * _NW) * _T1, _T1)]

        def dst(s):
            return blk_hbm.at[pl.ds((wid + s * _NW) * D, D), :]

        pltpu.async_copy(src(0), inb0, sem_i)

        def pair(p, carry):
            for par in range(2):
                s = p * 2 + par
                ib = bufs[par]

                @pl.when(s < n_my)
                def _():
                    pltpu.make_async_copy(src(0), ib, sem_i).wait()
                    pltpu.async_copy(ib, dst(s), sem_o)

                    @pl.when(s >= 1)
                    def _():
                        pltpu.make_async_copy(src(0), bufs[1 - par], sem_o).wait()

                    @pl.when(s + 1 < n_my)
                    def _():
                        pltpu.async_copy(src(s + 1), bufs[1 - par], sem_i)

            return carry

        lax.fori_loop(0, (n_my + 1) // 2, pair, 0)
        pltpu.make_async_copy(src(0), inb0, sem_o).wait()

    return detile


def _make_transpose(V, D):
    n_full = V // _T1
    tail = V - n_full * _T1
    mesh = plsc.VectorSubcoreMesh(core_axis_name="c", subcore_axis_name="s")

    @functools.partial(
        pl.kernel,
        mesh=mesh,
        out_type=jax.ShapeDtypeStruct((V, D), jnp.float32),
        scratch_types=[
            pltpu.VMEM((D, _T1), jnp.float32),
            pltpu.VMEM((D, _T1), jnp.float32),
            pltpu.VMEM((_T1, D), jnp.float32),
            pltpu.VMEM((_T1, D), jnp.float32),
            pltpu.VMEM((tail, D), jnp.float32),
            pltpu.SemaphoreType.DMA,
            pltpu.SemaphoreType.DMA,
        ],
        compiler_params=_SC_PARAMS,
    )
    def transpose(
        blk_hbm, tail_hbm, wr_hbm, inb0, inb1, outb0, outb1, tailb, sem_i, sem_o
    ):
        wid = lax.axis_index("s") * _NC + lax.axis_index("c")
        n_my = _n_my(wid, n_full)
        iota = lax.iota(jnp.int32, 16)
        iota16 = iota + 16
        bufs = ((inb0, outb0), (inb1, outb1))

        def src(s):
            return blk_hbm.at[pl.ds((wid + s * _NW) * D, D), :]

        def dst(s):
            return wr_hbm.at[pl.ds((wid + s * _NW) * _T1, _T1), :]

        pltpu.async_copy(src(0), inb0, sem_i)

        def pair(p, carry):
            for par in range(2):
                s = p * 2 + par
                ib, ob = bufs[par]

                @pl.when(s < n_my)
                def _():
                    pltpu.make_async_copy(src(0), ib, sem_i).wait()

                    @pl.when(s + 1 < n_my)
                    def _():
                        pltpu.async_copy(src(s + 1), bufs[1 - par][0], sem_i)

                    @pl.when(s >= 2)
                    def _():
                        pltpu.make_async_copy(src(0), ob, sem_o).wait()

                    def body(u, c):
                        t = u + iota
                        lo = plsc.load_gather(ib, [iota, t])
                        hi = plsc.load_gather(ib, [iota16, t])
                        plsc.store_scatter(ob, [t, iota], lo)
                        plsc.store_scatter(ob, [t, iota16], hi)
                        return c

                    lax.fori_loop(0, _T1 - 15, body, 0, unroll=8)

                    def body_wrap(u, c):
                        t0 = u + iota
                        t = jnp.where(t0 >= _T1, t0 - _T1, t0)
                        lo = plsc.load_gather(ib, [iota, t])
                        hi = plsc.load_gather(ib, [iota16, t])
                        plsc.store_scatter(ob, [t, iota], lo)
                        plsc.store_scatter(ob, [t, iota16], hi)
                        return c

                    lax.fori_loop(_T1 - 15, _T1, body_wrap, 0)
                    pltpu.async_copy(ob, dst(s), sem_o)

            return carry

        lax.fori_loop(0, (n_my + 1) // 2, pair, 0)
        pltpu.make_async_copy(src(0), outb0, sem_o).wait()
        pltpu.make_async_copy(src(0), outb1, sem_o).wait()

        if tail:
            @pl.when(wid == _NW - 1)
            def _():
                pltpu.sync_copy(tail_hbm, tailb)
                pltpu.sync_copy(tailb, wr_hbm.at[pl.ds(n_full * _T1, tail), :])

    return transpose


def _make_gather(V, D, L, B):
    n_win = B // _C2
    n_chunks = L * n_win // _NW
    seg = 8 * _C2
    mesh = plsc.VectorSubcoreMesh(core_axis_name="c", subcore_axis_name="s")

    @functools.partial(
        pl.kernel,
        mesh=mesh,
        out_type=jax.ShapeDtypeStruct(
            (L, D // 8, (B // 128) * 8 * 128), jnp.float32
        ),
        scratch_types=[
            pltpu.VMEM((_C2,), jnp.int32),
            pltpu.VMEM((_C2,), jnp.int32),
            pltpu.VMEM((_C2, D), jnp.float32),
            pltpu.VMEM((_C2, D), jnp.float32),
            pltpu.VMEM((D // 8, seg), jnp.float32),
            pltpu.SemaphoreType.DMA,
        ],
        compiler_params=_SC_PARAMS,
    )
    def gather(wr_hbm, idx_hbm, out_hbm, idxb0, idxb1, rows0, rows1, outt, sem):
        wid = lax.axis_index("s") * _NC + lax.axis_index("c")
        bw = (wid % n_win) * _C2
        l0 = wid // n_win
        iota = lax.iota(jnp.int32, 16)
        fvecs = [(c + iota) % D for c in range(D)]
        ftvecs = [fvecs[c] // 8 for c in range(D)]
        posvecs = [(fvecs[c] % 8) * 128 for c in range(D)]
        ib = (idxb0, idxb1)
        rb = (rows0, rows1)

        def idx_src(k):
            return idx_hbm.at[pl.ds((l0 + 2 * k) * B + bw, _C2)]

        pltpu.sync_copy(idx_src(0), idxb0)
        g = pltpu.async_copy(wr_hbm.at[idxb0], rows0, sem)

        for k in range(n_chunks):
            par = k & 1
            g.wait()
            if k + 1 < n_chunks:
                pltpu.sync_copy(idx_src(k + 1), ib[1 - par])
                g = pltpu.async_copy(wr_hbm.at[ib[1 - par]], rb[1 - par], sem)
            rows = rb[par]

            def tpose(b, c):
                j0 = b * 16
                jv = iota + j0
                jpart = ((jv >> 7) << 10) + (jv & 127)
                for d in range(D):
                    vals = plsc.load_gather(rows, [jv, fvecs[d]])
                    plsc.store_scatter(outt, [ftvecs[d], posvecs[d] + jpart], vals)
                return c

            lax.fori_loop(0, _C2 // 16, tpose, 0, unroll=2)
            pltpu.sync_copy(
                outt, out_hbm.at[l0 + 2 * k, :, pl.ds(bw * 8, seg)]
            )

    return gather


def kernel(word_indexes, W):
    B, L = word_indexes.shape
    V, D = W.shape
    wt = W.T
    n_full = V // _T1
    tail_rows = W[n_full * _T1:, :]
    idx_flat = word_indexes.T.astype(jnp.int32).reshape(B * L)
    blk = _make_detile(V, D)(wt)
    wr = _make_transpose(V, D)(blk, tail_rows)
    out3 = _make_gather(V, D, L, B)(wr, idx_flat)
    out5 = out3.reshape(L, D // 8, B // 128, 8, 128)
    return out5.transpose(2, 4, 0, 1, 3).reshape(B, L, D)
